# baseline (speedup 1.0000x reference)
import functools
import math

import jax
import jax.numpy as jnp
import numpy as np
from jax.experimental import pallas as pl
from jax.experimental.pallas import tpu as pltpu

B = 8; L = 512; CIN = 8; PL_ = 96; D = 1024; E = 8; K = 2; HID = 2048
PATCH = 16; STRIDE = 8; PAD = 8
NPAT = 64
GC = B * CIN
N = GC * NPAT
C = int(N * 1.25 * K / E)
BC = 256


def _pos_embed():
    pos = np.arange(NPAT, dtype=np.float32)[:, None]
    div = np.exp(np.arange(0, D, 2, dtype=np.float32) * -(math.log(10000.0) / D))
    pe = np.zeros((NPAT, D), dtype=np.float32)
    pe[:, 0::2] = np.sin(pos * div)
    pe[:, 1::2] = np.cos(pos * div)
    return jnp.asarray(pe)


def _prologue_body(xT_ref, wp_ref, wr_ref, pe_ref, X_ref, lg_ref, mu_ref, sd_ref):
    x = xT_ref[...]
    m = jnp.mean(x, axis=1, keepdims=True)
    xc = x - m
    v = jnp.mean(xc * xc, axis=1, keepdims=True)
    sd = jnp.sqrt(v + 1e-5)
    xn = xc / sd
    mu_ref[...] = m
    sd_ref[...] = sd
    xpad = jnp.concatenate(
        [xn, jnp.broadcast_to(xn[:, L - 1:L], (GC, PAD))], axis=1)
    wp = wp_ref[...].astype(jnp.bfloat16)
    wr = wr_ref[...].astype(jnp.bfloat16)
    for p in range(NPAT):
        seg = xpad[:, p * STRIDE: p * STRIDE + PATCH].astype(jnp.bfloat16)
        tok = jnp.dot(seg, wp, preferred_element_type=jnp.float32) + pe_ref[p]
        X_ref[:, p, :] = tok
        lg_ref[:, p, :] = jnp.dot(tok.astype(jnp.bfloat16), wr,
                                  preferred_element_type=jnp.float32)


def _prologue(xT, W_patch, W_r, pe):
    return pl.pallas_call(
        _prologue_body,
        out_shape=(
            jax.ShapeDtypeStruct((GC, NPAT, D), jnp.float32),
            jax.ShapeDtypeStruct((GC, NPAT, E), jnp.float32),
            jax.ShapeDtypeStruct((GC, 1), jnp.float32),
            jax.ShapeDtypeStruct((GC, 1), jnp.float32),
        ),
    )(xT, W_patch, W_r, pe)


def _routing_body(lg_ref, slotg_ref, slots_ref, w_ref, aux_ref):
    lg = lg_ref[...]
    m = jnp.max(lg, axis=1, keepdims=True)
    ex = jnp.exp(lg - m)
    s = jnp.sum(ex, axis=1, keepdims=True)
    probs = ex / s
    iota8 = jax.lax.broadcasted_iota(jnp.int32, (N, E), 1)
    p1 = jnp.max(probs, axis=1, keepdims=True)
    i1 = jnp.min(jnp.where(probs == p1, iota8, E), axis=1, keepdims=True)
    pm = jnp.where(iota8 == i1, -1.0, probs)
    p2 = jnp.max(pm, axis=1, keepdims=True)
    i2 = jnp.min(jnp.where(pm == p2, iota8, E), axis=1, keepdims=True)
    den = p1 + p2 + 1e-9
    g1 = p1 / den
    g2 = p2 / den
    A = ((iota8 == i1) | (iota8 == i2)).astype(jnp.float32)
    S = jnp.concatenate([jnp.zeros((1, E), jnp.float32), A[:-1]], axis=0)
    k = 1
    while k < N:
        S = S + jnp.concatenate(
            [jnp.zeros((k, E), jnp.float32), S[:-k]], axis=0)
        k *= 2
    pos1 = jnp.sum(jnp.where(iota8 == i1, S, 0.0), axis=1, keepdims=True)
    pos2 = jnp.sum(jnp.where(iota8 == i2, S, 0.0), axis=1, keepdims=True)
    keep1 = pos1 < C
    keep2 = pos2 < C
    slot1 = i1 * C + jnp.minimum(pos1, C - 1).astype(jnp.int32)
    slot2 = i2 * C + jnp.minimum(pos2, C - 1).astype(jnp.int32)
    slotg_ref[...] = jnp.concatenate([slot1, slot2], axis=1)
    slots_ref[...] = jnp.concatenate(
        [jnp.where(keep1, slot1, -1), jnp.where(keep2, slot2, -1)], axis=1)
    w_ref[...] = jnp.concatenate(
        [jnp.where(keep1, g1, 0.0), jnp.where(keep2, g2, 0.0)], axis=1)
    me = jnp.mean(probs, axis=0)
    ce = jnp.mean(A, axis=0)
    balance = 0.01 * E * jnp.sum(me * ce)
    lse = m[:, 0] + jnp.log(s[:, 0])
    zloss = 0.001 * jnp.mean(lse * lse)
    aux_ref[...] = jnp.broadcast_to(balance + zloss, (1, 1))


def _routing(logits):
    return pl.pallas_call(
        _routing_body,
        out_shape=(
            jax.ShapeDtypeStruct((N, K), jnp.int32),
            jax.ShapeDtypeStruct((N, K), jnp.int32),
            jax.ShapeDtypeStruct((N, K), jnp.float32),
            jax.ShapeDtypeStruct((1, 1), jnp.float32),
        ),
    )(logits)


def _ffn_body(x_ref, w1_ref, b1_ref, w2_ref, b2_ref, o_ref):
    x = x_ref[0].astype(jnp.bfloat16)
    w1 = w1_ref[0].astype(jnp.bfloat16)
    h = jnp.dot(x, w1, preferred_element_type=jnp.float32) + b1_ref[0]
    h = jax.nn.gelu(h).astype(jnp.bfloat16)
    w2 = w2_ref[0].astype(jnp.bfloat16)
    o_ref[0] = jnp.dot(h, w2, preferred_element_type=jnp.float32) + b2_ref[0]


def _expert_ffn(buf, W1, b1, W2, b2):
    return pl.pallas_call(
        _ffn_body,
        grid=(E, C // BC),
        in_specs=[
            pl.BlockSpec((1, BC, D), lambda e, i: (e, i, 0)),
            pl.BlockSpec((1, D, HID), lambda e, i: (e, 0, 0)),
            pl.BlockSpec((1, 1, HID), lambda e, i: (e, 0, 0)),
            pl.BlockSpec((1, HID, D), lambda e, i: (e, 0, 0)),
            pl.BlockSpec((1, 1, D), lambda e, i: (e, 0, 0)),
        ],
        out_specs=pl.BlockSpec((1, BC, D), lambda e, i: (e, i, 0)),
        out_shape=jax.ShapeDtypeStruct((E, C, D), jnp.float32),
    )(buf, W1, b1.reshape(E, 1, HID), W2, b2.reshape(E, 1, D))


def _head_body(y_ref, wh_ref, bh_ref, mu_ref, sd_ref, o_ref):
    p = pl.program_id(0)

    @pl.when(p == 0)
    def _():
        o_ref[...] = jnp.zeros_like(o_ref)

    yp = y_ref[:, 0, 0, :].astype(jnp.bfloat16)
    wh = wh_ref[:, 0, 0, :].astype(jnp.bfloat16)
    o_ref[...] += jnp.dot(yp, wh, preferred_element_type=jnp.float32)

    @pl.when(p == NPAT - 1)
    def _():
        acc = o_ref[...] + bh_ref[...]
        o_ref[...] = acc * sd_ref[...] + mu_ref[...]


def _head(y, W_head, b_head, mu, sd):
    y4 = y.reshape(GC, NPAT, 1, D)
    wh4 = W_head.reshape(D, NPAT, 1, PL_)
    return pl.pallas_call(
        _head_body,
        grid=(NPAT,),
        in_specs=[
            pl.BlockSpec((GC, 1, 1, D), lambda p: (0, p, 0, 0)),
            pl.BlockSpec((D, 1, 1, PL_), lambda p: (0, p, 0, 0)),
            pl.BlockSpec((1, PL_), lambda p: (0, 0)),
            pl.BlockSpec((GC, 1), lambda p: (0, 0)),
            pl.BlockSpec((GC, 1), lambda p: (0, 0)),
        ],
        out_specs=pl.BlockSpec((GC, PL_), lambda p: (0, 0)),
        out_shape=jax.ShapeDtypeStruct((GC, PL_), jnp.float32),
    )(y4, wh4, b_head.reshape(1, PL_), mu, sd)


def kernel(x_enc, x_mark_enc, x_dec, x_mark_dec, W_patch, W_r, W1, b1, W2, b2, W_head, b_head):
    xT = jnp.transpose(x_enc, (0, 2, 1)).reshape(GC, L)
    X3, lg3, mu, sd = _prologue(xT, W_patch, W_r, _pos_embed())
    X = X3.reshape(N, D)
    slotg, slots, w, aux = _routing(lg3.reshape(N, E))

    flat_g = slotg.reshape(-1)
    flat_e = flat_g // C
    pos_c = flat_g % C
    keep = (slots.reshape(-1) >= 0).astype(jnp.float32)
    x_rep = jnp.repeat(X, K, axis=0)
    buf = jnp.zeros((E, C, D), jnp.float32).at[flat_e, pos_c].add(
        x_rep * keep[:, None])

    yb = _expert_ffn(buf, W1, b1, W2, b2)

    y_pair = yb[flat_e, pos_c] * w.reshape(-1)[:, None]
    y = jnp.sum(y_pair.reshape(N, K, D), axis=1)

    dec_pre = _head(y, W_head, b_head, mu, sd)
    dec = dec_pre.reshape(B, CIN, PL_).transpose(0, 2, 1)
    return dec, aux.reshape(())
